# baseline (speedup 1.0000x reference)
import functools

import numpy as np
import jax
import jax.numpy as jnp
from jax import lax
from jax.experimental import pallas as pl
from jax.experimental.pallas import tpu as pltpu
from jax.experimental.pallas import tpu_sc as plsc

N_AT = 100000
NE = 119
D = 24
L = 16

NC = 2
NS = 16
NW = NC * NS

N_PAD = 100096
APW = 3200
APW_LAST = N_PAD - (NW - 1) * APW
N_SUB = 25
SUB = APW // N_SUB
SUB_CHUNKS = SUB // L
MAX_ASTART = N_AT - APW
IDX_ALLOC = APW + 2432


@functools.partial(
    pl.kernel,
    out_type=jax.ShapeDtypeStruct((D, N_PAD), jnp.float32),
    mesh=plsc.VectorSubcoreMesh(core_axis_name="c", subcore_axis_name="s"),
    scratch_types=[
        pltpu.VMEM((D, NE), jnp.float32),
        pltpu.VMEM((IDX_ALLOC,), jnp.int32),
        pltpu.VMEM((D, APW), jnp.float32),
        pltpu.SemaphoreType.DMA,
    ],
    compiler_params=pltpu.CompilerParams(needs_layout_passes=False),
)
def _gather_kernel(tab_hbm, idx_hbm, out_hbm, tab_v, idx_v, out_v, sem):
    cid = lax.axis_index("c")
    sid = lax.axis_index("s")
    wid = sid * NC + cid

    i0 = wid * APW
    astart = jnp.minimum(i0, MAX_ASTART)
    astart = pl.multiple_of(astart, 8)
    loff = i0 - astart

    ca = pltpu.async_copy(tab_hbm, tab_v, sem)
    cb = pltpu.async_copy(idx_hbm.at[pl.ds(astart, APW)], idx_v.at[pl.ds(0, APW)], sem)
    ca.wait()
    cb.wait()

    def _sub_block(b, carry):
        @plsc.parallel_loop(0, SUB_CHUNKS, 1, unroll=1)
        def _loop(c):
            cg = b * SUB_CHUNKS + c
            zvec = idx_v[pl.ds(loff + cg * L, L)]
            z = jnp.minimum(jnp.maximum(zvec, 0), NE - 1)
            for j in range(D):
                v = plsc.load_gather(tab_v, [jnp.full((L,), j, jnp.int32), z])
                out_v[j, pl.ds(cg * L, L)] = v

        dst = jnp.minimum(i0 + b * SUB, N_PAD - SUB)
        dst = pl.multiple_of(dst, 128)
        src = pl.multiple_of(dst - i0, 128)
        pltpu.async_copy(
            out_v.at[:, pl.ds(src, SUB)],
            out_hbm.at[:, pl.ds(dst, SUB)],
            sem,
        )
        return carry

    lax.fori_loop(0, N_SUB, _sub_block, 0)

    for _ in range(N_SUB):
        pltpu.make_async_copy(
            out_v.at[:, pl.ds(0, SUB)],
            out_hbm.at[:, pl.ds(0, SUB)],
            sem,
        ).wait()


@jax.jit
def kernel(atomic_numbers, e_config):
    return _gather_kernel(e_config.T, atomic_numbers).T[:N_AT]


# speedup vs baseline: 1.0886x; 1.0886x over previous
import functools

import numpy as np
import jax
import jax.numpy as jnp
from jax import lax
from jax.experimental import pallas as pl
from jax.experimental.pallas import tpu as pltpu
from jax.experimental.pallas import tpu_sc as plsc

N_AT = 100000
NE = 119
D = 24
L = 16

NC = 2
NS = 16
NW = NC * NS

N_PAD = 100096
APW = 3200
APW_LAST = N_PAD - (NW - 1) * APW
N_SUB = 5
SUB = APW // N_SUB
SUB_CHUNKS = SUB // L
MAX_ASTART = N_AT - APW
IDX_ALLOC = APW + 2432


@functools.partial(
    pl.kernel,
    out_type=jax.ShapeDtypeStruct((D, N_PAD), jnp.float32),
    mesh=plsc.VectorSubcoreMesh(core_axis_name="c", subcore_axis_name="s"),
    scratch_types=[
        pltpu.VMEM((D, NE), jnp.float32),
        pltpu.VMEM((IDX_ALLOC,), jnp.int32),
        pltpu.VMEM((D, APW), jnp.float32),
        pltpu.SemaphoreType.DMA,
    ],
    compiler_params=pltpu.CompilerParams(needs_layout_passes=False),
)
def _gather_kernel(tab_hbm, idx_hbm, out_hbm, tab_v, idx_v, out_v, sem):
    cid = lax.axis_index("c")
    sid = lax.axis_index("s")
    wid = sid * NC + cid

    i0 = wid * APW
    astart = jnp.minimum(i0, MAX_ASTART)
    astart = pl.multiple_of(astart, 8)
    loff = i0 - astart

    ca = pltpu.async_copy(tab_hbm, tab_v, sem)
    cb = pltpu.async_copy(idx_hbm.at[pl.ds(astart, APW)], idx_v.at[pl.ds(0, APW)], sem)
    ca.wait()
    cb.wait()

    def _sub_block(b, carry):
        @plsc.parallel_loop(0, SUB_CHUNKS, 1, unroll=1)
        def _loop(c):
            cg = b * SUB_CHUNKS + c
            zvec = idx_v[pl.ds(loff + cg * L, L)]
            z = jnp.minimum(jnp.maximum(zvec, 0), NE - 1)
            for j in range(D):
                v = plsc.load_gather(tab_v, [jnp.full((L,), j, jnp.int32), z])
                out_v[j, pl.ds(cg * L, L)] = v

        dst = jnp.minimum(i0 + b * SUB, N_PAD - SUB)
        dst = pl.multiple_of(dst, 128)
        src = pl.multiple_of(dst - i0, 128)
        pltpu.async_copy(
            out_v.at[:, pl.ds(src, SUB)],
            out_hbm.at[:, pl.ds(dst, SUB)],
            sem,
        )
        return carry

    lax.fori_loop(0, N_SUB, _sub_block, 0)

    for _ in range(N_SUB):
        pltpu.make_async_copy(
            out_v.at[:, pl.ds(0, SUB)],
            out_hbm.at[:, pl.ds(0, SUB)],
            sem,
        ).wait()


@jax.jit
def kernel(atomic_numbers, e_config):
    return _gather_kernel(e_config.T, atomic_numbers).T[:N_AT]
